# baseline (speedup 1.0000x reference)
import functools

import numpy as np
import jax
import jax.numpy as jnp
from jax import lax
from jax.experimental import pallas as pl
from jax.experimental.pallas import tpu as pltpu
from jax.experimental.pallas import tpu_sc as plsc

_D = 64
_L = 200
_B = 1024
_NC = 2
_NS = 16
_NW = _NC * _NS
_LBLK = 8
_NBB = _B // 128
_NUNITS = (_L // _LBLK) * _NBB
_UPW = -(-_NUNITS // _NW)


def _pos_encoding(length, depth):
    positions = np.arange(length).reshape(-1, 1)
    depths = np.array([2 * (i // 2) for i in range(depth)]).reshape(1, -1)
    angle_rates = 1.0 / 10000 ** (depths / depth)
    angles = positions * angle_rates
    encoding = np.cos(angles)
    encoding[:, ::2] = np.sin(encoding[:, ::2])
    return encoding.astype(np.float32)


_ENC128 = np.zeros((_L, 128), np.float32)
_ENC128[:, :_D] = _pos_encoding(_L, _D)
_ENC128 = jnp.asarray(_ENC128)

_mesh = plsc.VectorSubcoreMesh(core_axis_name="c", subcore_axis_name="s")


@functools.partial(
    pl.kernel,
    mesh=_mesh,
    out_type=jax.ShapeDtypeStruct((_L, _D, _B), jnp.float32),
    scratch_types=[
        pltpu.VMEM((_L, 128), jnp.float32),
        pltpu.VMEM((_LBLK, 128), jnp.int32),
        pltpu.VMEM((_LBLK, 128), jnp.int32),
        pltpu.VMEM((_LBLK, 128), jnp.int32),
        [pltpu.VMEM((128, 128), jnp.float32) for _ in range(2)],
        [pltpu.VMEM((_D, 128), jnp.float32) for _ in range(2)],
        pltpu.SemaphoreType.DMA((2,)),
        pltpu.SemaphoreType.DMA((2,)),
    ],
    compiler_params=pltpu.CompilerParams(use_tc_tiling_on_sc=True,
                                         needs_layout_passes=False),
)
def _emb_kernel(t128_hbm, xT_hbm, enc_hbm, out_hbm,
                enc_v, xb, idx_p, hoff, pk, slab, g_sem, s_sem):
    wid = lax.axis_index("s") * _NC + lax.axis_index("c")
    pltpu.sync_copy(enc_hbm, enc_v)

    toks = [lax.iota(jnp.int32, 16) + 16 * c for c in range(8)]

    def g_issue(li, i):
        pltpu.async_copy(t128_hbm.at[idx_p.at[li]], pk[i], g_sem.at[i])

    def g_wait(i):
        pltpu.make_async_copy(t128_hbm.at[idx_p.at[0]], pk[i],
                              g_sem.at[i]).wait()

    def s_issue(l, b0, i):
        pltpu.async_copy(slab[i], out_hbm.at[l, :, pl.ds(b0, 128)],
                         s_sem.at[i])

    def s_wait(i):
        pltpu.make_async_copy(slab[i], out_hbm.at[0, :, pl.ds(0, 128)],
                              s_sem.at[i]).wait()

    def unit(k, carry):
        u = wid + k * _NW

        @pl.when(u < _NUNITS)
        def _do_unit():
            l_blk = u // _NBB
            b_blk = u % _NBB
            l0 = l_blk * _LBLK
            b0 = b_blk * 128
            pltpu.sync_copy(xT_hbm.at[pl.ds(l0, _LBLK), pl.ds(b0, 128)], xb)
            for li in range(_LBLK):
                for c in range(8):
                    cs = pl.ds(16 * c, 16)
                    v = xb[li, cs]
                    idx_p[li, cs] = v >> 1
                    hoff[li, cs] = (v & 1) << 6
            g_issue(0, 0)
            for li in range(_LBLK):
                i = li % 2
                if li + 1 < _LBLK:
                    g_issue(li + 1, 1 - i)
                g_wait(i)
                l = l0 + li
                lsp = jnp.full((16,), l, jnp.int32)
                hvecs = [hoff[li, pl.ds(16 * c, 16)] for c in range(8)]
                if li >= 2:
                    s_wait(i)

                def sel(d, c2):
                    dsp = jnp.full((16,), d, jnp.int32)
                    encs = plsc.load_gather(enc_v, [lsp, dsp])
                    for c in range(8):
                        gv = plsc.load_gather(pk[i], [toks[c], hvecs[c] + dsp])
                        slab[i][d, pl.ds(16 * c, 16)] = gv + encs
                    return c2

                lax.fori_loop(0, _D, sel, 0)
                s_issue(l, b0, i)
            s_wait(0)
            s_wait(1)

        return carry

    lax.fori_loop(0, _UPW, unit, 0)


@jax.jit
def kernel(x, table):
    t128 = table.reshape(500000, 128)
    xT = x.T.astype(jnp.int32)
    out = _emb_kernel(t128, xT, _ENC128)
    return out.transpose(2, 0, 1)


# speedup vs baseline: 1.1249x; 1.1249x over previous
import functools

import numpy as np
import jax
import jax.numpy as jnp
from jax import lax
from jax.experimental import pallas as pl
from jax.experimental.pallas import tpu as pltpu
from jax.experimental.pallas import tpu_sc as plsc

_D = 64
_L = 200
_B = 1024
_NC = 2
_NS = 16
_NW = _NC * _NS
_LBLK = 8
_NBB = _B // 128
_NUNITS = (_L // _LBLK) * _NBB
_UPW = -(-_NUNITS // _NW)


def _pos_encoding(length, depth):
    positions = np.arange(length).reshape(-1, 1)
    depths = np.array([2 * (i // 2) for i in range(depth)]).reshape(1, -1)
    angle_rates = 1.0 / 10000 ** (depths / depth)
    angles = positions * angle_rates
    encoding = np.cos(angles)
    encoding[:, ::2] = np.sin(encoding[:, ::2])
    return encoding.astype(np.float32)


_ENC_SLAB = jnp.asarray(
    np.broadcast_to(_pos_encoding(_L, _D)[:, :, None], (_L, _D, 128)).copy())

_mesh = plsc.VectorSubcoreMesh(core_axis_name="c", subcore_axis_name="s")


@functools.partial(
    pl.kernel,
    mesh=_mesh,
    out_type=jax.ShapeDtypeStruct((_L, _D, _B), jnp.float32),
    scratch_types=[
        pltpu.VMEM((_LBLK, 128), jnp.int32),
        pltpu.VMEM((_LBLK, 128), jnp.int32),
        pltpu.VMEM((_LBLK, 128), jnp.int32),
        [pltpu.VMEM((128, 128), jnp.float32) for _ in range(2)],
        [pltpu.VMEM((_D, 128), jnp.float32) for _ in range(2)],
        pltpu.SemaphoreType.DMA((2,)),
        pltpu.SemaphoreType.DMA((2,)),
    ],
    compiler_params=pltpu.CompilerParams(use_tc_tiling_on_sc=True,
                                         needs_layout_passes=False),
)
def _emb_kernel(t128_hbm, xT_hbm, enc_hbm, out_hbm,
                xb, idx_p, hoff, pk, slab, g_sem, s_sem):
    wid = lax.axis_index("s") * _NC + lax.axis_index("c")

    toks = [lax.iota(jnp.int32, 16) + 16 * c for c in range(8)]

    def g_issue(li, i):
        pltpu.async_copy(t128_hbm.at[idx_p.at[li]], pk[i], g_sem.at[i])

    def g_wait(i):
        pltpu.make_async_copy(t128_hbm.at[idx_p.at[0]], pk[i],
                              g_sem.at[i]).wait()

    def s_issue(l, b0, i):
        pltpu.async_copy(slab[i], out_hbm.at[l, :, pl.ds(b0, 128)],
                         s_sem.at[i])

    def s_wait(i):
        pltpu.make_async_copy(slab[i], out_hbm.at[0, :, pl.ds(0, 128)],
                              s_sem.at[i]).wait()

    def unit(k, carry):
        u = wid + k * _NW

        @pl.when(u < _NUNITS)
        def _do_unit():
            l_blk = u // _NBB
            b_blk = u % _NBB
            l0 = l_blk * _LBLK
            b0 = b_blk * 128
            pltpu.sync_copy(xT_hbm.at[pl.ds(l0, _LBLK), pl.ds(b0, 128)], xb)
            for li in range(_LBLK):
                for c in range(8):
                    cs = pl.ds(16 * c, 16)
                    v = xb[li, cs]
                    idx_p[li, cs] = v >> 1
                    hoff[li, cs] = (v & 1) << 6
            g_issue(0, 0)
            for li in range(_LBLK):
                i = li % 2
                if li + 1 < _LBLK:
                    g_issue(li + 1, 1 - i)
                l = l0 + li
                hvecs = [hoff[li, pl.ds(16 * c, 16)] for c in range(8)]
                if li >= 2:
                    s_wait(i)
                pltpu.sync_copy(enc_hbm.at[l], slab[i])
                g_wait(i)

                @plsc.parallel_loop(0, _D, unroll=2)
                def _sel(d):
                    dsp = jnp.full((16,), d, jnp.int32)
                    for c in range(8):
                        gv = plsc.load_gather(pk[i], [toks[c], hvecs[c] + dsp])
                        plsc.addupdate(slab[i].at[d, pl.ds(16 * c, 16)], gv)

                s_issue(l, b0, i)
            s_wait(0)
            s_wait(1)

        return carry

    lax.fori_loop(0, _UPW, unit, 0)


@jax.jit
def kernel(x, table):
    t128 = table.reshape(500000, 128)
    xT = x.T.astype(jnp.int32)
    out = _emb_kernel(t128, xT, _ENC_SLAB)
    return out.transpose(2, 0, 1)


# speedup vs baseline: 1.3040x; 1.1593x over previous
import functools

import numpy as np
import jax
import jax.numpy as jnp
from jax import lax
from jax.experimental import pallas as pl
from jax.experimental.pallas import tpu as pltpu
from jax.experimental.pallas import tpu_sc as plsc

_D = 64
_L = 200
_B = 1024
_NC = 2
_NS = 16
_NW = _NC * _NS
_EPW = _B // _NW
_NBUF = 8
_GROUPS = _EPW // _NBUF
_CA = 104
_CB = _L - _CA


def _pos_encoding(length, depth):
    positions = np.arange(length).reshape(-1, 1)
    depths = np.array([2 * (i // 2) for i in range(depth)]).reshape(1, -1)
    angle_rates = 1.0 / 10000 ** (depths / depth)
    angles = positions * angle_rates
    encoding = np.cos(angles)
    encoding[:, ::2] = np.sin(encoding[:, ::2])
    return encoding.astype(np.float32)


_ENC = jnp.asarray(_pos_encoding(_L, _D))

_mesh = plsc.VectorSubcoreMesh(core_axis_name="c", subcore_axis_name="s")


@functools.partial(
    pl.kernel,
    mesh=_mesh,
    out_type=jax.ShapeDtypeStruct((_B * _L, _D), jnp.float32),
    scratch_types=[
        pltpu.VMEM((_EPW * _L,), jnp.int32),
        [pltpu.VMEM((_L, _D), jnp.float32) for _ in range(_NBUF)],
        pltpu.VMEM_SHARED((_L, _D), jnp.float32),
        pltpu.SemaphoreType.DMA((_NBUF,)),
        pltpu.SemaphoreType.DMA((_NBUF,)),
        pltpu.SemaphoreType.DMA((_NBUF,)),
    ],
    compiler_params=pltpu.CompilerParams(use_tc_tiling_on_sc=False),
)
def _emb_kernel(table_hbm, xf_hbm, enc_hbm, out_hbm,
                idx_all, rows, enc_sh, pre_sem, g_sem, st_sem):
    sid = lax.axis_index("s")
    wid = sid * _NC + lax.axis_index("c")
    base_row = wid * _EPW * _L

    pltpu.sync_copy(xf_hbm.at[pl.ds(base_row, _EPW * _L)], idx_all)

    @pl.when(sid == 0)
    def _stage_enc():
        pltpu.sync_copy(enc_hbm, enc_sh)

    plsc.subcore_barrier()

    def pre_issue(b):
        pltpu.async_copy(enc_sh, rows[b], pre_sem.at[b])

    def pre_wait(b):
        pltpu.make_async_copy(enc_sh, rows[b], pre_sem.at[b]).wait()

    def g_issue(le, b):
        o = le * _L
        pltpu.async_copy(table_hbm.at[idx_all.at[pl.ds(o, _CA)]],
                         rows[b].at[pl.ds(0, _CA)], g_sem.at[b], add=True)
        pltpu.async_copy(table_hbm.at[idx_all.at[pl.ds(o + _CA, _CB)]],
                         rows[b].at[pl.ds(_CA, _CB)], g_sem.at[b], add=True)

    def g_wait(b):
        pltpu.make_async_copy(table_hbm.at[idx_all.at[pl.ds(0, _CA)]],
                              rows[b].at[pl.ds(0, _CA)], g_sem.at[b]).wait()
        pltpu.make_async_copy(table_hbm.at[idx_all.at[pl.ds(_CA, _CB)]],
                              rows[b].at[pl.ds(_CA, _CB)], g_sem.at[b]).wait()

    def st_issue(le, b):
        pltpu.async_copy(rows[b], out_hbm.at[pl.ds(base_row + le * _L, _L)],
                         st_sem.at[b])

    def st_wait(b):
        pltpu.make_async_copy(rows[b], out_hbm.at[pl.ds(0, _L)],
                              st_sem.at[b]).wait()

    for b in range(_NBUF):
        pre_issue(b)
    for b in range(_NBUF // 2):
        pre_wait(b)
        g_issue(b, b)

    def group(g, carry):
        le0 = _NBUF * g
        for b in range(4):
            g_wait(b)
            st_issue(le0 + b, b)
        for b in range(4, 8):
            pre_wait(b)
            g_issue(le0 + b, b)
        for b in range(4):
            st_wait(b)
            pre_issue(b)
        for b in range(4, 8):
            g_wait(b)
            st_issue(le0 + b, b)

        @pl.when(g < _GROUPS - 1)
        def _next_gathers():
            for b in range(4):
                pre_wait(b)
                g_issue(le0 + _NBUF + b, b)

        for b in range(4, 8):
            st_wait(b)
            pre_issue(b)
        return carry

    lax.fori_loop(0, _GROUPS, group, 0)

    for b in range(_NBUF):
        pre_wait(b)


@jax.jit
def kernel(x, table):
    xf = x.reshape(-1).astype(jnp.int32)
    out = _emb_kernel(table, xf, _ENC)
    return out.reshape(_B, _L, _D)
